# baseline (speedup 1.0000x reference)
import functools

import jax
import jax.numpy as jnp
from jax import lax
from jax.experimental import pallas as pl
from jax.experimental.pallas import tpu as pltpu


def _fused_xattn_kernel(x_ref, data_ref, wq_ref, bq_ref, wkv_ref, bkv_ref,
                        wp_ref, bp_ref, o_ref, kv_sc, *, heads, attn_ch,
                        width, bp_n):
    n_data = data_ref.shape[1]
    slab = 4 * attn_ch
    for b in range(bp_n):
        kv = jnp.dot(data_ref[b], wkv_ref[...],
                     preferred_element_type=jnp.float32)
        kvb = (kv + bkv_ref[...]).astype(jnp.bfloat16)
        kv_sc[b, :, 0:width] = kvb[:, 0:width]
        for g in range(heads // 2):
            base = width + g * slab
            kv_sc[b, :, base:base + 2 * attn_ch] = (
                kvb[:, width + g * 2 * attn_ch:width + (g + 1) * 2 * attn_ch])
            kv_sc[b, :, base + 2 * attn_ch:base + slab] = jnp.ones(
                (n_data, 2 * attn_ch), jnp.bfloat16)

    for b in range(bp_n):
        q = (jnp.dot(x_ref[b], wq_ref[...], preferred_element_type=jnp.float32)
             + bq_ref[...]).astype(jnp.bfloat16)
        outs = []
        for h in range(heads):
            hs = h * attn_ch
            q_h = q[:, hs:hs + attn_ch]
            k_h = kv_sc[b, :, hs:hs + attn_ch]
            vs = width + (h // 2) * slab
            v_g = kv_sc[b, :, vs:vs + slab]
            s = lax.dot_general(q_h, k_h, (((1,), (1,)), ((), ())),
                                preferred_element_type=jnp.float32)
            s = s.astype(jnp.bfloat16)
            m = jnp.max(s, axis=-1, keepdims=True)
            p = jnp.exp2(s - m)
            o_w = jnp.dot(p, v_g, preferred_element_type=jnp.float32)
            l_w = o_w[:, 2 * attn_ch:3 * attn_ch]
            ls = (h % 2) * attn_ch
            outs.append(o_w[:, ls:ls + attn_ch] / l_w)

        o = jnp.concatenate(outs, axis=-1).astype(jnp.bfloat16)
        o_ref[b] = (jnp.dot(o, wp_ref[...], preferred_element_type=jnp.float32)
                    + bp_ref[...])


def kernel(c_q_w, c_q_b, c_kv_w, c_kv_b, c_proj_w, c_proj_b, x, data):
    bs, n_ctx, width = x.shape
    _, n_data, data_width = data.shape
    heads = 8
    attn_ch = width // heads
    scale2 = 1.4426950408889634 / (attn_ch ** 0.5)

    idx = jnp.arange(width)
    h_idx = idx // attn_ch
    c_idx = idx % attn_ch
    k_cols = h_idx * (2 * attn_ch) + c_idx
    perm = jnp.concatenate([k_cols, k_cols + attn_ch])
    wkv = c_kv_w[:, perm].astype(jnp.bfloat16)
    bkv = c_kv_b[perm].reshape(1, 2 * width)

    wq = (c_q_w * scale2).astype(jnp.bfloat16)
    bq = (c_q_b * scale2).reshape(1, width)
    wp = c_proj_w.astype(jnp.bfloat16)
    bp = c_proj_b.reshape(1, width)
    xb = x.astype(jnp.bfloat16)
    db = data.astype(jnp.bfloat16)

    bp_n = 4 if bs % 4 == 0 else (2 if bs % 2 == 0 else 1)
    kv_n = width + (heads // 2) * 4 * attn_ch
    kern = functools.partial(_fused_xattn_kernel, heads=heads,
                             attn_ch=attn_ch, width=width, bp_n=bp_n)
    out = pl.pallas_call(
        kern,
        out_shape=jax.ShapeDtypeStruct((bs, n_ctx, width), jnp.float32),
        grid=(bs // bp_n,),
        in_specs=[
            pl.BlockSpec((bp_n, n_ctx, width), lambda i: (i, 0, 0)),
            pl.BlockSpec((bp_n, n_data, data_width), lambda i: (i, 0, 0)),
            pl.BlockSpec((width, width), lambda i: (0, 0)),
            pl.BlockSpec((1, width), lambda i: (0, 0)),
            pl.BlockSpec((data_width, 2 * width), lambda i: (0, 0)),
            pl.BlockSpec((1, 2 * width), lambda i: (0, 0)),
            pl.BlockSpec((width, width), lambda i: (0, 0)),
            pl.BlockSpec((1, width), lambda i: (0, 0)),
        ],
        out_specs=pl.BlockSpec((bp_n, n_ctx, width), lambda i: (i, 0, 0)),
        scratch_shapes=[pltpu.VMEM((bp_n, n_data, kv_n), jnp.bfloat16)],
        compiler_params=pltpu.CompilerParams(
            dimension_semantics=("parallel",)
        ),
    )(xb, db, wq, bq, wkv, bkv, wp, bp)
    return out


# speedup vs baseline: 1.3064x; 1.3064x over previous
import functools

import jax
import jax.numpy as jnp
from jax import lax
from jax.experimental import pallas as pl
from jax.experimental.pallas import tpu as pltpu


def _fused_xattn_kernel(x_ref, data_ref, wq_ref, bq_ref, wkv_ref, bkv_ref,
                        wp_ref, bp_ref, o_ref, kv_sc, *, heads, attn_ch,
                        width, bp_n):
    n_data = data_ref.shape[1]
    slab = 4 * attn_ch
    for b in range(bp_n):
        kv = jnp.dot(data_ref[b], wkv_ref[...],
                     preferred_element_type=jnp.float32)
        kvb = (kv + bkv_ref[...]).astype(jnp.bfloat16)
        kv_sc[b, :, 0:width] = kvb[:, 0:width]
        for g in range(heads // 2):
            base = width + g * slab
            kv_sc[b, :, base:base + 2 * attn_ch] = (
                kvb[:, width + g * 2 * attn_ch:width + (g + 1) * 2 * attn_ch])
            kv_sc[b, :, base + 2 * attn_ch:base + slab] = jnp.ones(
                (n_data, 2 * attn_ch), jnp.bfloat16)

    qs = [(jnp.dot(x_ref[b], wq_ref[...], preferred_element_type=jnp.float32)
           + bq_ref[...]).astype(jnp.bfloat16) for b in range(bp_n)]
    outs = [[] for _ in range(bp_n)]
    for h in range(heads):
        for b in range(bp_n):
            hs = h * attn_ch
            q_h = qs[b][:, hs:hs + attn_ch]
            k_h = kv_sc[b, :, hs:hs + attn_ch]
            vs = width + (h // 2) * slab
            v_g = kv_sc[b, :, vs:vs + slab]
            s = lax.dot_general(q_h, k_h, (((1,), (1,)), ((), ())),
                                preferred_element_type=jnp.float32)
            s = s.astype(jnp.bfloat16)
            m = jnp.max(s, axis=-1, keepdims=True)
            p = jnp.exp2(s - m)
            o_w = jnp.dot(p, v_g, preferred_element_type=jnp.float32)
            l_w = o_w[:, 2 * attn_ch:3 * attn_ch]
            ls = (h % 2) * attn_ch
            outs[b].append(o_w[:, ls:ls + attn_ch] / l_w)

    for b in range(bp_n):
        o = jnp.concatenate(outs[b], axis=-1).astype(jnp.bfloat16)
        o_ref[b] = (jnp.dot(o, wp_ref[...], preferred_element_type=jnp.float32)
                    + bp_ref[...])


def kernel(c_q_w, c_q_b, c_kv_w, c_kv_b, c_proj_w, c_proj_b, x, data):
    bs, n_ctx, width = x.shape
    _, n_data, data_width = data.shape
    heads = 8
    attn_ch = width // heads
    scale2 = 1.4426950408889634 / (attn_ch ** 0.5)

    idx = jnp.arange(width)
    h_idx = idx // attn_ch
    c_idx = idx % attn_ch
    k_cols = h_idx * (2 * attn_ch) + c_idx
    perm = jnp.concatenate([k_cols, k_cols + attn_ch])
    wkv = c_kv_w[:, perm].astype(jnp.bfloat16)
    bkv = c_kv_b[perm].reshape(1, 2 * width)

    wq = (c_q_w * scale2).astype(jnp.bfloat16)
    bq = (c_q_b * scale2).reshape(1, width)
    wp = c_proj_w.astype(jnp.bfloat16)
    bp = c_proj_b.reshape(1, width)
    xb = x.astype(jnp.bfloat16)
    db = data.astype(jnp.bfloat16)

    bp_n = 2 if bs % 2 == 0 else 1
    kv_n = width + (heads // 2) * 4 * attn_ch
    kern = functools.partial(_fused_xattn_kernel, heads=heads,
                             attn_ch=attn_ch, width=width, bp_n=bp_n)
    out = pl.pallas_call(
        kern,
        out_shape=jax.ShapeDtypeStruct((bs, n_ctx, width), jnp.float32),
        grid=(bs // bp_n,),
        in_specs=[
            pl.BlockSpec((bp_n, n_ctx, width), lambda i: (i, 0, 0)),
            pl.BlockSpec((bp_n, n_data, data_width), lambda i: (i, 0, 0)),
            pl.BlockSpec((width, width), lambda i: (0, 0)),
            pl.BlockSpec((1, width), lambda i: (0, 0)),
            pl.BlockSpec((data_width, 2 * width), lambda i: (0, 0)),
            pl.BlockSpec((1, 2 * width), lambda i: (0, 0)),
            pl.BlockSpec((width, width), lambda i: (0, 0)),
            pl.BlockSpec((1, width), lambda i: (0, 0)),
        ],
        out_specs=pl.BlockSpec((bp_n, n_ctx, width), lambda i: (i, 0, 0)),
        scratch_shapes=[pltpu.VMEM((bp_n, n_data, kv_n), jnp.bfloat16)],
        compiler_params=pltpu.CompilerParams(
            dimension_semantics=("parallel",)
        ),
    )(xb, db, wq, bq, wkv, bkv, wp, bp)
    return out
